# Initial kernel scaffold; baseline (speedup 1.0000x reference)
import jax
import jax.numpy as jnp
from jax.experimental import pallas as pl


def kernel(title_ids, title_tokens, title_table, token_table):
    raise NotImplementedError("write your pallas kernel here")


# speedup vs baseline: 14.0429x; 14.0429x over previous
import functools

import jax
import jax.numpy as jnp
from jax import lax
from jax.experimental import pallas as pl
from jax.experimental.pallas import tpu as pltpu
from jax.experimental.pallas import tpu_sc as plsc

B = 16384
L = 20
D = 32
NC, NS, LANES = 2, 16, 16
NW = NC * NS
BPW = B // NW
RB = 64
NB = BPW // RB
NG = BPW // LANES


def _body(ids_hbm, tok_hbm, ttab_hbm, ktab_hbm, out_hbm,
          ids_v, tok_v, e1_v, out_v, gat_v, p_v, q_v, t0_v, sem, gsem, osem):
    wid = lax.axis_index("s") * NC + lax.axis_index("c")
    base = wid * BPW

    pltpu.sync_copy(ids_hbm.at[pl.ds(base, BPW)], ids_v)
    pltpu.sync_copy(tok_hbm.at[:, pl.ds(base, BPW)], tok_v)
    pltpu.sync_copy(ktab_hbm.at[pl.ds(0, 8)], t0_v)

    e1_copies = []
    for j in range(BPW // 128):
        c = pltpu.async_copy(
            ttab_hbm.at[ids_v.at[pl.ds(j * 128, 128)]],
            e1_v.at[pl.ds(j * 128, 128)], gsem)
        e1_copies.append(c)

    @pl.loop(0, NG)
    def _(g):
        s = g * LANES
        cnt = jnp.zeros((LANES,), jnp.float32)
        for l in range(L):
            t = tok_v[l, pl.ds(s, LANES)]
            cnt = cnt + (t != 0).astype(jnp.float32)
        inv = 1.0 / jnp.maximum(cnt, 1.0)
        p_v[pl.ds(s, LANES)] = inv
        q_v[pl.ds(s, LANES)] = (float(L) - cnt) * inv

    for c in e1_copies:
        c.wait()

    for jb in range(NB):
        r0 = jb * RB
        copies = [
            pltpu.async_copy(
                ktab_hbm.at[tok_v.at[l, pl.ds(r0, RB)]], gat_v.at[l], sem)
            for l in range(L)
        ]
        for c in copies:
            c.wait()

        @pl.loop(0, RB)
        def _(rr, r0=r0):
            r = r0 + rr
            a0 = gat_v[0, rr, pl.ds(0, LANES)]
            a1 = gat_v[0, rr, pl.ds(LANES, LANES)]
            for l in range(1, L):
                a0 = a0 + gat_v[l, rr, pl.ds(0, LANES)]
                a1 = a1 + gat_v[l, rr, pl.ds(LANES, LANES)]
            out_v[r, pl.ds(0, LANES)] = e1_v[r, pl.ds(0, LANES)]
            out_v[r, pl.ds(LANES, LANES)] = e1_v[r, pl.ds(LANES, LANES)]
            ridx = jnp.full((LANES,), r, jnp.int32)
            p = plsc.load_gather(p_v, [ridx])
            q = plsc.load_gather(q_v, [ridx])
            t0a = t0_v[0, pl.ds(0, LANES)]
            t0b = t0_v[0, pl.ds(LANES, LANES)]
            out_v[r, pl.ds(D, LANES)] = a0 * p - q * t0a
            out_v[r, pl.ds(D + LANES, LANES)] = a1 * p - q * t0b

    pltpu.sync_copy(out_v, out_hbm.at[pl.ds(base, BPW)])


@jax.jit
def kernel(title_ids, title_tokens, title_table, token_table):
    tokens_t = title_tokens.T.astype(jnp.int32)
    mesh = plsc.VectorSubcoreMesh(core_axis_name="c", subcore_axis_name="s")
    k = pl.kernel(
        _body,
        out_type=jax.ShapeDtypeStruct((B, 2 * D), jnp.float32),
        mesh=mesh,
        compiler_params=pltpu.CompilerParams(
            use_tc_tiling_on_sc=False, needs_layout_passes=False),
        scratch_types=[
            pltpu.VMEM((BPW,), jnp.int32),
            pltpu.VMEM((L, BPW), jnp.int32),
            pltpu.VMEM((BPW, D), jnp.float32),
            pltpu.VMEM((BPW, 2 * D), jnp.float32),
            pltpu.VMEM((L, RB, D), jnp.float32),
            pltpu.VMEM((BPW,), jnp.float32),
            pltpu.VMEM((BPW,), jnp.float32),
            pltpu.VMEM((8, D), jnp.float32),
            pltpu.SemaphoreType.DMA,
            pltpu.SemaphoreType.DMA,
            pltpu.SemaphoreType.DMA,
        ],
    )
    return k(title_ids.astype(jnp.int32), tokens_t, title_table, token_table)


# speedup vs baseline: 14.9415x; 1.0640x over previous
import jax
import jax.numpy as jnp
from jax import lax
from jax.experimental import pallas as pl
from jax.experimental.pallas import tpu as pltpu
from jax.experimental.pallas import tpu_sc as plsc

B = 16384
L = 20
D = 32
NC, NS, LANES = 2, 16, 16
NW = NC * NS
BPW = B // NW
NTC = 128
RB = 32
CPB = RB * L // NTC
NB = BPW // RB
NG = BPW // LANES


def _body(ids_hbm, tok_hbm, ttab_hbm, ktab_hbm, out_hbm,
          ids_v, tok_v, e1_v, out_v, gat_v, p_v, q_v, t0_v, sems, gsem):
    wid = lax.axis_index("s") * NC + lax.axis_index("c")
    base = wid * BPW

    pltpu.sync_copy(ids_hbm.at[pl.ds(base, BPW)], ids_v)
    pltpu.sync_copy(tok_hbm.at[pl.ds(base * L, BPW * L)], tok_v)
    pltpu.sync_copy(ktab_hbm.at[pl.ds(0, 8)], t0_v)

    e1_copies = [
        pltpu.async_copy(
            ttab_hbm.at[ids_v.at[pl.ds(j * NTC, NTC)]],
            e1_v.at[pl.ds(j * NTC, NTC)], gsem)
        for j in range(BPW // NTC)
    ]

    def fire(jb, buf):
        t0 = jb * RB * L
        return [
            pltpu.async_copy(
                ktab_hbm.at[tok_v.at[pl.ds(t0 + c * NTC, NTC)]],
                gat_v.at[buf, pl.ds(c * NTC, NTC)], sems.at[buf])
            for c in range(CPB)
        ]

    inflight = [fire(0, 0), fire(1, 1)]

    lane20 = lax.iota(jnp.int32, LANES) * L

    @pl.loop(0, NG)
    def _(g):
        s = g * LANES
        idx0 = s * L + lane20
        cnt = jnp.zeros((LANES,), jnp.float32)
        for l in range(L):
            t = plsc.load_gather(tok_v, [idx0 + l])
            cnt = cnt + (t != 0).astype(jnp.float32)
        inv = 1.0 / jnp.maximum(cnt, 1.0)
        p_v[pl.ds(s, LANES)] = inv
        q_v[pl.ds(s, LANES)] = (float(L) - cnt) * inv

    for c in e1_copies:
        c.wait()

    for jb in range(NB):
        buf = jb % 3
        for c in inflight[jb]:
            c.wait()
        if jb + 2 < NB:
            inflight.append(fire(jb + 2, (jb + 2) % 3))
        else:
            inflight.append([])

        @pl.loop(0, RB)
        def _(rr, jb=jb, buf=buf):
            r = jb * RB + rr
            rowb = rr * L
            a0 = gat_v[buf, rowb, pl.ds(0, LANES)]
            a1 = gat_v[buf, rowb, pl.ds(LANES, LANES)]
            for l in range(1, L):
                a0 = a0 + gat_v[buf, rowb + l, pl.ds(0, LANES)]
                a1 = a1 + gat_v[buf, rowb + l, pl.ds(LANES, LANES)]
            out_v[r, pl.ds(0, LANES)] = e1_v[r, pl.ds(0, LANES)]
            out_v[r, pl.ds(LANES, LANES)] = e1_v[r, pl.ds(LANES, LANES)]
            ridx = jnp.full((LANES,), r, jnp.int32)
            p = plsc.load_gather(p_v, [ridx])
            q = plsc.load_gather(q_v, [ridx])
            t0a = t0_v[0, pl.ds(0, LANES)]
            t0b = t0_v[0, pl.ds(LANES, LANES)]
            out_v[r, pl.ds(D, LANES)] = a0 * p - q * t0a
            out_v[r, pl.ds(D + LANES, LANES)] = a1 * p - q * t0b

    pltpu.sync_copy(out_v, out_hbm.at[pl.ds(base, BPW)])


@jax.jit
def kernel(title_ids, title_tokens, title_table, token_table):
    tokens_flat = title_tokens.reshape(-1).astype(jnp.int32)
    mesh = plsc.VectorSubcoreMesh(core_axis_name="c", subcore_axis_name="s")
    k = pl.kernel(
        _body,
        out_type=jax.ShapeDtypeStruct((B, 2 * D), jnp.float32),
        mesh=mesh,
        compiler_params=pltpu.CompilerParams(
            use_tc_tiling_on_sc=False, needs_layout_passes=False),
        scratch_types=[
            pltpu.VMEM((BPW,), jnp.int32),
            pltpu.VMEM((BPW * L,), jnp.int32),
            pltpu.VMEM((BPW, D), jnp.float32),
            pltpu.VMEM((BPW, 2 * D), jnp.float32),
            pltpu.VMEM((3, RB * L, D), jnp.float32),
            pltpu.VMEM((BPW,), jnp.float32),
            pltpu.VMEM((BPW,), jnp.float32),
            pltpu.VMEM((8, D), jnp.float32),
            pltpu.SemaphoreType.DMA((3,)),
            pltpu.SemaphoreType.DMA,
        ],
    )
    return k(title_ids.astype(jnp.int32), tokens_flat, title_table, token_table)
